# Initial kernel scaffold; baseline (speedup 1.0000x reference)
import functools

import jax
import jax.numpy as jnp
from jax import lax
from jax.experimental import pallas as pl
from jax.experimental.pallas import tpu as pltpu
from jax.experimental.pallas import tpu_sc as plsc

LANES = 16
CHUNK = 80


def _linear_body(x_ref, w_ref, b_ref, o_ref):
    o_ref[...] = lax.dot_general(
        x_ref[...], w_ref[...], (((1,), (1,)), ((), ())),
        preferred_element_type=jnp.float32) + b_ref[...][None, :]


def _linear(x, W, b):
    n, _ = x.shape
    h = W.shape[0]
    return pl.pallas_call(
        _linear_body,
        out_shape=jax.ShapeDtypeStruct((n, h), jnp.float32),
    )(x, W, b)


def _edge_agg_body(nchunks_pw, n_nodes,
                   h1_hbm, srcm_hbm, dstm_hbm, acc_hbm, deg_hbm,
                   rows_v, src_v, dst_v, ones_v, zbuf_v, zflat_v,
                   acc_sh, deg_sh, sem):
    cid = lax.axis_index("c")
    sid = lax.axis_index("s")
    n_sub = 16
    wid = cid * n_sub + sid
    rows_per_sub = n_nodes // n_sub
    zrows = zbuf_v.shape[0]

    def _z2(i, _):
        r = i // (128 // LANES)
        c = i % (128 // LANES)
        zbuf_v[r, pl.ds(c * LANES, LANES)] = jnp.zeros((LANES,), jnp.float32)
        return 0
    lax.fori_loop(0, zrows * (128 // LANES), _z2, 0)

    def _z1(i, _):
        zflat_v[pl.ds(i * LANES, LANES)] = jnp.zeros((LANES,), jnp.float32)
        return 0
    lax.fori_loop(0, zflat_v.shape[0] // LANES, _z1, 0)

    def _o1(i, _):
        ones_v[pl.ds(i * LANES, LANES)] = jnp.ones((LANES,), jnp.float32)
        return 0
    lax.fori_loop(0, CHUNK // LANES, _o1, 0)

    def _zacc(k, _):
        pltpu.sync_copy(zbuf_v, acc_sh.at[pl.ds(sid * rows_per_sub + k * zrows, zrows)])
        return 0
    lax.fori_loop(0, rows_per_sub // zrows, _zacc, 0)

    zf = zflat_v.shape[0]

    @pl.when(sid == 0)
    def _zdeg():
        def _zd(k, _):
            pltpu.sync_copy(zflat_v, deg_sh.at[pl.ds(k * zf, zf)])
            return 0
        lax.fori_loop(0, n_nodes // zf, _zd, 0)

    plsc.subcore_barrier()

    base = wid * nchunks_pw
    pltpu.sync_copy(srcm_hbm.at[pl.ds(base, nchunks_pw)], src_v)
    pltpu.sync_copy(dstm_hbm.at[pl.ds(base, nchunks_pw)], dst_v)

    def _edge(j, _):
        pltpu.async_copy(h1_hbm.at[src_v.at[j]], rows_v, sem).wait()
        pltpu.sync_copy(rows_v, acc_sh.at[dst_v.at[j]], add=True)
        pltpu.sync_copy(ones_v, deg_sh.at[src_v.at[j]], add=True)
        return 0
    lax.fori_loop(0, nchunks_pw, _edge, 0)

    plsc.subcore_barrier()

    def _wacc(k, _):
        off = sid * rows_per_sub + k * zrows
        pltpu.sync_copy(acc_sh.at[pl.ds(off, zrows)], acc_hbm.at[cid, pl.ds(off, zrows)])
        return 0
    lax.fori_loop(0, rows_per_sub // zrows, _wacc, 0)

    @pl.when(sid == 0)
    def _wdeg():
        def _wd(k, _):
            pltpu.sync_copy(deg_sh.at[pl.ds(k * zf, zf)], deg_hbm.at[cid, pl.ds(k * zf, zf)])
            return 0
        lax.fori_loop(0, n_nodes // zf, _wd, 0)


def _edge_aggregate(h1, srcm, dstm, n_nodes):
    nchunks = srcm.shape[0]
    n_workers = 32
    nchunks_pw = nchunks // n_workers
    mesh = plsc.VectorSubcoreMesh(core_axis_name="c", subcore_axis_name="s")
    kern = pl.kernel(
        functools.partial(_edge_agg_body, nchunks_pw, n_nodes),
        out_type=(
            jax.ShapeDtypeStruct((2, n_nodes, 128), jnp.float32),
            jax.ShapeDtypeStruct((2, n_nodes), jnp.float32),
        ),
        mesh=mesh,
        scratch_types=(
            pltpu.VMEM((CHUNK, 128), jnp.float32),
            pltpu.VMEM((nchunks_pw, CHUNK), jnp.int32),
            pltpu.VMEM((nchunks_pw, CHUNK), jnp.int32),
            pltpu.VMEM((CHUNK,), jnp.float32),
            pltpu.VMEM((125, 128), jnp.float32),
            pltpu.VMEM((400,), jnp.float32),
            pltpu.VMEM_SHARED((n_nodes, 128), jnp.float32),
            pltpu.VMEM_SHARED((n_nodes,), jnp.float32),
            pltpu.SemaphoreType.DMA,
        ),
    )
    return kern(h1, srcm, dstm)


def _combine_body(n_edges, acc_ref, deg_ref, w2_ref, b2_ref, o_ref):
    agg = acc_ref[0] + acc_ref[1]
    t = jnp.tanh(agg)
    w2s = jnp.sum(w2_ref[...], axis=0)
    deg = deg_ref[0] + deg_ref[1]
    total = jnp.sum(jnp.sum(t * w2s[None, :], axis=1) * deg)
    total = total + n_edges * jnp.sum(b2_ref[...])
    o_ref[0, 0] = total


def _combine(acc, deg, W2, b2, n_edges):
    return pl.pallas_call(
        functools.partial(_combine_body, float(n_edges)),
        out_shape=jax.ShapeDtypeStruct((1, 1), jnp.float32),
    )(acc, deg, W2, b2)


def kernel(x, edge_index, W1, b1, W2, b2):
    n_nodes = x.shape[0]
    n_edges = edge_index.shape[1]
    src = edge_index[0].reshape(n_edges // CHUNK, CHUNK)
    dst = edge_index[1].reshape(n_edges // CHUNK, CHUNK)

    h1 = _linear(x, W1, b1)
    acc, deg = _edge_aggregate(h1, src, dst, n_nodes)
    return _combine(acc, deg, W2, b2, n_edges)


# speedup vs baseline: 9.0819x; 9.0819x over previous
import functools

import jax
import jax.numpy as jnp
from jax import lax
from jax.experimental import pallas as pl
from jax.experimental.pallas import tpu as pltpu
from jax.experimental.pallas import tpu_sc as plsc

LANES = 16
CHUNK = 80
HALF = 64
N_SUB = 16


def _linear_body(x_ref, w_ref, b_ref, o_ref):
    h = lax.dot_general(
        x_ref[...], w_ref[...], (((1,), (1,)), ((), ())),
        preferred_element_type=jnp.float32) + b_ref[...][None, :]
    o_ref[0] = h[:, :HALF]
    o_ref[1] = h[:, HALF:]


def _linear_split(x, W, b):
    n, _ = x.shape
    return pl.pallas_call(
        _linear_body,
        out_shape=jax.ShapeDtypeStruct((2, n, HALF), jnp.float32),
    )(x, W, b)


def _edge_agg_body(nchunks_ps, n_nodes,
                   h1_hbm, srcm_hbm, dstm_hbm, acc_hbm, deg_hbm,
                   rows_v, src_v, dst_v, ones_v, zbuf_v, zflat_v,
                   acc_sh, deg_sh, sem):
    cid = lax.axis_index("c")
    sid = lax.axis_index("s")
    zrows = zbuf_v.shape[0]
    nzchunks = n_nodes // zrows
    zk = (nzchunks + N_SUB - 1) // N_SUB
    zf = zflat_v.shape[0]

    def _z2(i, _):
        r = i // (HALF // LANES)
        c = i % (HALF // LANES)
        zbuf_v[r, pl.ds(c * LANES, LANES)] = jnp.zeros((LANES,), jnp.float32)
        return 0
    lax.fori_loop(0, zrows * (HALF // LANES), _z2, 0)

    def _z1(i, _):
        zflat_v[pl.ds(i * LANES, LANES)] = jnp.zeros((LANES,), jnp.float32)
        return 0
    lax.fori_loop(0, zf // LANES, _z1, 0)

    def _o1(i, _):
        ones_v[pl.ds(i * LANES, LANES)] = jnp.ones((LANES,), jnp.float32)
        return 0
    lax.fori_loop(0, CHUNK // LANES, _o1, 0)

    def _zacc(k, _):
        j = sid + k * N_SUB
        @pl.when(j < nzchunks)
        def _():
            pltpu.sync_copy(zbuf_v, acc_sh.at[pl.ds(j * zrows, zrows)])
        return 0
    lax.fori_loop(0, zk, _zacc, 0)

    @pl.when((sid == 0) & (cid == 0))
    def _zdeg():
        def _zd(k, _):
            pltpu.sync_copy(zflat_v, deg_sh.at[pl.ds(k * zf, zf)])
            return 0
        lax.fori_loop(0, n_nodes // zf, _zd, 0)

    pltpu.sync_copy(srcm_hbm.at[sid], src_v)
    pltpu.sync_copy(dstm_hbm.at[sid], dst_v)

    plsc.subcore_barrier()

    def _edge(j, _):
        pltpu.async_copy(h1_hbm.at[cid].at[src_v.at[j]], rows_v, sem).wait()
        pltpu.sync_copy(rows_v, acc_sh.at[dst_v.at[j]], add=True)
        @pl.when(cid == 0)
        def _():
            pltpu.sync_copy(ones_v, deg_sh.at[src_v.at[j]], add=True)
        return 0
    lax.fori_loop(0, nchunks_ps, _edge, 0)

    plsc.subcore_barrier()

    def _wacc(k, _):
        j = sid + k * N_SUB
        @pl.when(j < nzchunks)
        def _():
            off = j * zrows
            pltpu.sync_copy(acc_sh.at[pl.ds(off, zrows)], zbuf_v)
            pltpu.sync_copy(zbuf_v, acc_hbm.at[cid, pl.ds(off, zrows)])
        return 0
    lax.fori_loop(0, zk, _wacc, 0)

    @pl.when((sid == 0) & (cid == 0))
    def _wdeg():
        def _wd(k, _):
            pltpu.sync_copy(deg_sh.at[pl.ds(k * zf, zf)], zflat_v)
            pltpu.sync_copy(zflat_v, deg_hbm.at[pl.ds(k * zf, zf)])
            return 0
        lax.fori_loop(0, n_nodes // zf, _wd, 0)


def _edge_aggregate(h1s, srcm, dstm, n_nodes):
    nchunks_ps = srcm.shape[1]
    mesh = plsc.VectorSubcoreMesh(core_axis_name="c", subcore_axis_name="s")
    kern = pl.kernel(
        functools.partial(_edge_agg_body, nchunks_ps, n_nodes),
        out_type=(
            jax.ShapeDtypeStruct((2, n_nodes, HALF), jnp.float32),
            jax.ShapeDtypeStruct((n_nodes,), jnp.float32),
        ),
        mesh=mesh,
        compiler_params=pltpu.CompilerParams(use_tc_tiling_on_sc=False),
        scratch_types=(
            pltpu.VMEM((CHUNK, HALF), jnp.float32),
            pltpu.VMEM((nchunks_ps, CHUNK), jnp.int32),
            pltpu.VMEM((nchunks_ps, CHUNK), jnp.int32),
            pltpu.VMEM((CHUNK,), jnp.float32),
            pltpu.VMEM((200, HALF), jnp.float32),
            pltpu.VMEM((400,), jnp.float32),
            pltpu.VMEM_SHARED((n_nodes, HALF), jnp.float32),
            pltpu.VMEM_SHARED((n_nodes,), jnp.float32),
            pltpu.SemaphoreType.DMA,
        ),
    )
    return kern(h1s, srcm, dstm)


def _combine_body(n_edges, acc_ref, deg_ref, w2_ref, b2_ref, o_ref):
    w2s = jnp.sum(w2_ref[...], axis=0)
    deg = deg_ref[...]
    ta = jnp.tanh(acc_ref[0])
    tb = jnp.tanh(acc_ref[1])
    row = jnp.sum(ta * w2s[None, :HALF], axis=1) + jnp.sum(tb * w2s[None, HALF:], axis=1)
    total = jnp.sum(row * deg) + n_edges * jnp.sum(b2_ref[...])
    o_ref[...] = total[None, None]


def _combine(acc, deg, W2, b2, n_edges):
    return pl.pallas_call(
        functools.partial(_combine_body, float(n_edges)),
        out_shape=jax.ShapeDtypeStruct((1, 1), jnp.float32),
    )(acc, deg, W2, b2)


def kernel(x, edge_index, W1, b1, W2, b2):
    n_nodes = x.shape[0]
    n_edges = edge_index.shape[1]
    src = edge_index[0].reshape(N_SUB, n_edges // (N_SUB * CHUNK), CHUNK)
    dst = edge_index[1].reshape(N_SUB, n_edges // (N_SUB * CHUNK), CHUNK)

    h1s = _linear_split(x, W1, b1)
    acc, deg = _edge_aggregate(h1s, src, dst, n_nodes)
    return _combine(acc, deg, W2, b2, n_edges)


# speedup vs baseline: 14.7143x; 1.6202x over previous
import functools

import jax
import jax.numpy as jnp
from jax import lax
from jax.experimental import pallas as pl
from jax.experimental.pallas import tpu as pltpu
from jax.experimental.pallas import tpu_sc as plsc

LANES = 16
CHUNK = 80
HALF = 64
N_SUB = 16


def _linear_body(x_ref, w_ref, b_ref, o_ref):
    h = lax.dot_general(
        x_ref[...], w_ref[...], (((1,), (1,)), ((), ())),
        preferred_element_type=jnp.float32) + b_ref[...][None, :]
    o_ref[0] = h[:, :HALF]
    o_ref[1] = h[:, HALF:]


def _linear_split(x, W, b):
    n, _ = x.shape
    return pl.pallas_call(
        _linear_body,
        out_shape=jax.ShapeDtypeStruct((2, n, HALF), jnp.float32),
    )(x, W, b)


def _edge_agg_body(nchunks_ps, n_nodes,
                   h1_hbm, srcm_hbm, dstm_hbm, acc_hbm, deg_hbm,
                   rows0_v, rows1_v, src_v, dst_v, zbuf_v, degl_v,
                   acc_sh, semg0, semg1):
    cid = lax.axis_index("c")
    sid = lax.axis_index("s")
    zrows = zbuf_v.shape[0]
    nzchunks = n_nodes // zrows
    zk = (nzchunks + N_SUB - 1) // N_SUB

    def _z2(i, _):
        r = i // (HALF // LANES)
        c = i % (HALF // LANES)
        zbuf_v[r, pl.ds(c * LANES, LANES)] = jnp.zeros((LANES,), jnp.float32)
        return 0
    lax.fori_loop(0, zrows * (HALF // LANES), _z2, 0)

    def _zacc(k, _):
        j = sid + k * N_SUB
        @pl.when(j < nzchunks)
        def _():
            pltpu.sync_copy(zbuf_v, acc_sh.at[pl.ds(j * zrows, zrows)])
        return 0
    lax.fori_loop(0, zk, _zacc, 0)

    pltpu.sync_copy(srcm_hbm.at[sid], src_v)
    pltpu.sync_copy(dstm_hbm.at[sid], dst_v)

    def _zd(i, _):
        degl_v[pl.ds(i * LANES, LANES)] = jnp.zeros((LANES,), jnp.float32)
        return 0
    lax.fori_loop(0, n_nodes // LANES, _zd, 0)

    ones16 = jnp.ones((LANES,), jnp.float32)
    vpc = CHUNK // LANES
    def _hist(i, _):
        idx = src_v[i // vpc, pl.ds((i % vpc) * LANES, LANES)]
        plsc.addupdate_scatter(degl_v, [idx], ones16)
        return 0
    lax.fori_loop(0, nchunks_ps * vpc, _hist, 0)
    pltpu.sync_copy(degl_v,
                    deg_hbm.at[pl.ds((cid * N_SUB + sid) * n_nodes, n_nodes)])

    plsc.subcore_barrier()

    table = h1_hbm.at[cid]
    pltpu.async_copy(table.at[src_v.at[0]], rows0_v, semg0)

    def _edge(i, _):
        j0 = 2 * i
        d1 = pltpu.async_copy(table.at[src_v.at[j0 + 1]], rows1_v, semg1)
        pltpu.make_async_copy(table.at[src_v.at[j0]], rows0_v, semg0).wait()
        pltpu.sync_copy(rows0_v, acc_sh.at[dst_v.at[j0]], add=True)
        jn = jnp.where(j0 + 2 < nchunks_ps, j0 + 2, 0)
        pltpu.async_copy(table.at[src_v.at[jn]], rows0_v, semg0)
        d1.wait()
        pltpu.sync_copy(rows1_v, acc_sh.at[dst_v.at[j0 + 1]], add=True)
        return 0
    lax.fori_loop(0, nchunks_ps // 2, _edge, 0)
    pltpu.make_async_copy(table.at[src_v.at[0]], rows0_v, semg0).wait()

    plsc.subcore_barrier()

    def _wacc(k, _):
        j = sid + k * N_SUB
        @pl.when(j < nzchunks)
        def _():
            off = j * zrows
            pltpu.sync_copy(acc_sh.at[pl.ds(off, zrows)], zbuf_v)
            pltpu.sync_copy(zbuf_v, acc_hbm.at[cid, pl.ds(off, zrows)])
        return 0
    lax.fori_loop(0, zk, _wacc, 0)


def _edge_aggregate(h1s, srcm, dstm, n_nodes):
    nchunks_ps = srcm.shape[1]
    mesh = plsc.VectorSubcoreMesh(core_axis_name="c", subcore_axis_name="s")
    kern = pl.kernel(
        functools.partial(_edge_agg_body, nchunks_ps, n_nodes),
        out_type=(
            jax.ShapeDtypeStruct((2, n_nodes, HALF), jnp.float32),
            jax.ShapeDtypeStruct((2 * N_SUB * n_nodes,), jnp.float32),
        ),
        mesh=mesh,
        compiler_params=pltpu.CompilerParams(use_tc_tiling_on_sc=False,
                                             needs_layout_passes=False),
        scratch_types=(
            pltpu.VMEM((CHUNK, HALF), jnp.float32),
            pltpu.VMEM((CHUNK, HALF), jnp.float32),
            pltpu.VMEM((nchunks_ps, CHUNK), jnp.int32),
            pltpu.VMEM((nchunks_ps, CHUNK), jnp.int32),
            pltpu.VMEM((200, HALF), jnp.float32),
            pltpu.VMEM((n_nodes,), jnp.float32),
            pltpu.VMEM_SHARED((n_nodes, HALF), jnp.float32),
            pltpu.SemaphoreType.DMA,
            pltpu.SemaphoreType.DMA,
        ),
    )
    return kern(h1s, srcm, dstm)


def _combine_body(n_edges, acc_ref, deg_ref, w2_ref, b2_ref, o_ref):
    w2s = jnp.sum(w2_ref[...], axis=0)
    deg = 0.5 * jnp.sum(deg_ref[...], axis=0)
    ta = jnp.tanh(acc_ref[0])
    tb = jnp.tanh(acc_ref[1])
    row = jnp.sum(ta * w2s[None, :HALF], axis=1) + jnp.sum(tb * w2s[None, HALF:], axis=1)
    total = jnp.sum(row * deg) + n_edges * jnp.sum(b2_ref[...])
    o_ref[...] = total[None, None]


def _combine(acc, deg, W2, b2, n_edges):
    return pl.pallas_call(
        functools.partial(_combine_body, float(n_edges)),
        out_shape=jax.ShapeDtypeStruct((1, 1), jnp.float32),
    )(acc, deg, W2, b2)


def kernel(x, edge_index, W1, b1, W2, b2):
    n_nodes = x.shape[0]
    n_edges = edge_index.shape[1]
    src = edge_index[0].reshape(N_SUB, n_edges // (N_SUB * CHUNK), CHUNK)
    dst = edge_index[1].reshape(N_SUB, n_edges // (N_SUB * CHUNK), CHUNK)

    h1s = _linear_split(x, W1, b1)
    acc, deg = _edge_aggregate(h1s, src, dst, n_nodes)
    return _combine(acc, deg.reshape(2 * N_SUB, n_nodes), W2, b2, n_edges)
